# baseline (speedup 1.0000x reference)
import functools

import jax
import jax.numpy as jnp
from jax import lax
from jax.experimental import pallas as pl
from jax.experimental.pallas import tpu as pltpu
from jax.experimental.pallas import tpu_sc as plsc

N_NODES_PAD = 100352
CHUNK = 128
LANES = 16


def _fem_body(tab_hbm, gmap_hbm, smap_hbm, zeros_hbm, dummy_hbm, dummy2_hbm,
              out_hbm, *, n_cells, cpt, n_chunks, gidx_v, sidx_v, gbuf,
              vbuf, out_buf, acc_sh, tab_sh, sem, ssem):
    nc = 2
    ns = 16
    cid = lax.axis_index("c")
    sid = lax.axis_index("s")
    wid = sid * nc + cid

    glen = (n_chunks + 1) * 3 * CHUNK
    for v in range(3):
        pltpu.sync_copy(gmap_hbm.at[v, wid], gidx_v.at[pl.ds(v * glen, glen)])
        pltpu.sync_copy(smap_hbm.at[v, wid], sidx_v.at[v])

    slice_n = N_NODES_PAD // ns
    for part in range(3):
        psl = pl.ds(part * N_NODES_PAD + sid * slice_n, slice_n)
        pltpu.sync_copy(tab_hbm.at[psl], tab_sh.at[psl])
    pltpu.sync_copy(zeros_hbm, acc_sh.at[pl.ds(sid * slice_n, slice_n)])
    plsc.subcore_barrier()

    lane_iota = lax.iota(jnp.int32, LANES)
    nanc = jnp.full((LANES,), jnp.nan, jnp.float32)
    zeroc = jnp.zeros((LANES,), jnp.float32)
    halfc = jnp.full((LANES,), 0.5, jnp.float32)
    tile_base = wid * cpt

    def issue(ch, parity):
        for v in range(3):
            ids = gidx_v.at[pl.ds(v * glen + ch * (3 * CHUNK), 3 * CHUNK)]
            dst = gbuf.at[pl.ds(parity * (9 * CHUNK) + v * (3 * CHUNK),
                                3 * CHUNK)]
            pltpu.async_copy(tab_sh.at[ids], dst, sem)

    def drain_gathers(parity):
        pltpu.make_async_copy(
            dummy_hbm, gbuf.at[pl.ds(parity * (9 * CHUNK), 9 * CHUNK)],
            sem).wait()

    def drain_scatters(parity):
        pltpu.make_async_copy(dummy2_hbm, vbuf.at[parity], ssem).wait()

    issue(0, 0)

    def chunk_body(ch, carry):
        p = lax.rem(ch, 2)
        drain_gathers(p)
        issue(ch + 1, 1 - p)

        @pl.when(ch >= 2)
        def _():
            drain_scatters(p)

        pbase = p * (9 * CHUNK)
        for s in range(CHUNK // LANES):
            sl = pl.ds(s * LANES, LANES)

            def gld(v, field):
                return gbuf[pl.ds(pbase + v * (3 * CHUNK) + field * CHUNK
                                  + s * LANES, LANES)]

            xi = gld(0, 0)
            yi = gld(0, 1)
            ui = gld(0, 2)
            xj = gld(1, 0)
            yj = gld(1, 1)
            uj = gld(1, 2)
            xk = gld(2, 0)
            yk = gld(2, 1)
            uk = gld(2, 2)

            bi = yj - yk
            bj = yk - yi
            bk = yi - yj
            ci = xk - xj
            cj = xi - xk
            ck = xj - xi
            det = ck * bj - cj * bk
            inv = halfc / jnp.abs(det)
            inv = jnp.where(det == zeroc, nanc, inv)
            g1 = (bi * ui + bj * uj + bk * uk) * inv
            g2 = (ci * ui + cj * uj + ck * uk) * inv
            vi = -(bi * g1 + ci * g2)
            vj = -(bj * g1 + cj * g2)
            vk = -(bk * g1 + ck * g2)
            r = lane_iota + s * LANES
            valid = (tile_base + ch * CHUNK + r) < n_cells
            vi = jnp.where(valid, vi, zeroc)
            vj = jnp.where(valid, vj, zeroc)
            vk = jnp.where(valid, vk, zeroc)
            vbuf[p, 0, sl] = vi
            vbuf[p, 1, sl] = vj
            vbuf[p, 2, sl] = vk

        pltpu.async_copy(vbuf.at[p, 0], acc_sh.at[sidx_v.at[0, ch]], ssem, add=True)
        pltpu.async_copy(vbuf.at[p, 1], acc_sh.at[sidx_v.at[1, ch]], ssem, add=True)
        pltpu.async_copy(vbuf.at[p, 2], acc_sh.at[sidx_v.at[2, ch]], ssem, add=True)
        return carry

    lax.fori_loop(0, n_chunks, chunk_body, 0, unroll=False)
    drain_gathers(n_chunks % 2)
    drain_scatters((n_chunks - 2) % 2)
    drain_scatters((n_chunks - 1) % 2)

    plsc.subcore_barrier()
    pltpu.sync_copy(acc_sh.at[pl.ds(sid * slice_n, slice_n)], out_buf)
    pltpu.sync_copy(out_buf, out_hbm.at[cid, pl.ds(sid * slice_n, slice_n)])


def _build_sc_call(n_cells, cpt, n_chunks):
    mesh = plsc.VectorSubcoreMesh(core_axis_name="c", subcore_axis_name="s")
    slice_n = N_NODES_PAD // 16
    body = functools.partial(_fem_body, n_cells=n_cells, cpt=cpt,
                             n_chunks=n_chunks)
    return pl.kernel(
        body,
        out_type=jax.ShapeDtypeStruct((2, N_NODES_PAD), jnp.float32),
        mesh=mesh,
        scratch_types=dict(
            gidx_v=pltpu.VMEM((3 * (n_chunks + 1) * 3 * CHUNK,), jnp.int32),
            sidx_v=pltpu.VMEM((3, n_chunks, CHUNK), jnp.int32),
            gbuf=pltpu.VMEM((2 * 9 * CHUNK,), jnp.float32),
            vbuf=pltpu.VMEM((2, 3, CHUNK), jnp.float32),
            out_buf=pltpu.VMEM((slice_n,), jnp.float32),
            acc_sh=pltpu.VMEM_SHARED((N_NODES_PAD,), jnp.float32),
            tab_sh=pltpu.VMEM_SHARED((3 * N_NODES_PAD,), jnp.float32),
            sem=pltpu.SemaphoreType.DMA,
            ssem=pltpu.SemaphoreType.DMA,
        ),
    )


def kernel(mesh_points, u, cell_node_map):
    n = mesh_points.shape[0]
    t = cell_node_map.shape[0]
    n_tiles = 32
    cpt = -(-t // (n_tiles * CHUNK)) * CHUNK
    n_chunks = cpt // CHUNK
    slice_n = N_NODES_PAD // 16

    mp = mesh_points.astype(jnp.float32)
    npad = N_NODES_PAD - n
    tab = jnp.concatenate([
        jnp.pad(mp[:, 0], (0, npad)),
        jnp.pad(mp[:, 1], (0, npad)),
        jnp.pad(u.astype(jnp.float32), (0, npad)),
    ])

    idx = cell_node_map.astype(jnp.int32).T
    pad = n_tiles * cpt - t
    idx = jnp.pad(idx, ((0, 0), (0, pad)))
    sidx = idx.reshape(3, n_tiles, n_chunks, CHUNK)
    gidx = jnp.concatenate(
        [sidx, sidx + N_NODES_PAD, sidx + 2 * N_NODES_PAD], axis=3)
    gidx = jnp.concatenate(
        [gidx, jnp.zeros((3, n_tiles, 1, 3 * CHUNK), jnp.int32)], axis=2)
    gidx = gidx.reshape(3, n_tiles, (n_chunks + 1) * 3 * CHUNK)
    zeros = jnp.zeros((slice_n,), jnp.float32)
    dummy = jnp.zeros((9 * CHUNK,), jnp.float32)
    dummy2 = jnp.zeros((3, CHUNK), jnp.float32)

    call = _build_sc_call(t, cpt, n_chunks)
    partials = call(tab, gidx, sidx, zeros, dummy, dummy2)
    out = partials[0] + partials[1]
    return out[:n]


# speedup vs baseline: 1.1886x; 1.1886x over previous
import functools

import jax
import jax.numpy as jnp
from jax import lax
from jax.experimental import pallas as pl
from jax.experimental.pallas import tpu as pltpu
from jax.experimental.pallas import tpu_sc as plsc

N_NODES_PAD = 100352
CHUNK = 128
LANES = 16


def _fem_body(xs_hbm, ys_hbm, us_hbm, cmap_hbm, zeros_hbm, dummy_hbm, out_hbm,
              *, n_cells, cpt, n_chunks, idx_v, gbuf,
              vbuf, out_buf, acc_sh, xs_sh, ys_sh, us_sh, sem, ssem):
    nc = 2
    ns = 16
    cid = lax.axis_index("c")
    sid = lax.axis_index("s")
    wid = sid * nc + cid

    pltpu.sync_copy(cmap_hbm.at[0, wid], idx_v.at[0])
    pltpu.sync_copy(cmap_hbm.at[1, wid], idx_v.at[1])
    pltpu.sync_copy(cmap_hbm.at[2, wid], idx_v.at[2])

    slice_n = N_NODES_PAD // ns
    tsl = pl.ds(sid * slice_n, slice_n)
    pltpu.sync_copy(xs_hbm.at[tsl], xs_sh.at[tsl])
    pltpu.sync_copy(ys_hbm.at[tsl], ys_sh.at[tsl])
    pltpu.sync_copy(us_hbm.at[tsl], us_sh.at[tsl])
    pltpu.sync_copy(zeros_hbm, acc_sh.at[tsl])
    plsc.subcore_barrier()

    lane_iota = lax.iota(jnp.int32, LANES)
    nanc = jnp.full((LANES,), jnp.nan, jnp.float32)
    zeroc = jnp.zeros((LANES,), jnp.float32)
    halfc = jnp.full((LANES,), 0.5, jnp.float32)
    tile_base = wid * cpt
    tabs = (xs_sh, ys_sh, us_sh)

    def issue(ch, parity):
        for v in range(3):
            ids = idx_v.at[v, ch]
            for tb in range(3):
                pltpu.async_copy(tabs[tb].at[ids], gbuf.at[parity, v, tb], sem)

    def drain_gathers(parity):
        pltpu.make_async_copy(dummy_hbm, gbuf.at[parity], sem).wait()

    def drain_scatters(parity):
        pltpu.make_async_copy(dummy_hbm.at[0], vbuf.at[parity], ssem).wait()

    issue(0, 0)

    def chunk_body(ch, carry):
        p = lax.rem(ch, 2)
        drain_gathers(p)
        issue(ch + 1, 1 - p)

        @pl.when(ch >= 2)
        def _():
            drain_scatters(p)

        for s in range(CHUNK // LANES):
            sl = pl.ds(s * LANES, LANES)
            xi = gbuf[p, 0, 0, sl]
            yi = gbuf[p, 0, 1, sl]
            ui = gbuf[p, 0, 2, sl]
            xj = gbuf[p, 1, 0, sl]
            yj = gbuf[p, 1, 1, sl]
            uj = gbuf[p, 1, 2, sl]
            xk = gbuf[p, 2, 0, sl]
            yk = gbuf[p, 2, 1, sl]
            uk = gbuf[p, 2, 2, sl]

            bi = yj - yk
            bj = yk - yi
            bk = yi - yj
            ci = xk - xj
            cj = xi - xk
            ck = xj - xi
            det = ck * bj - cj * bk
            inv = halfc / jnp.abs(det)
            inv = jnp.where(det == zeroc, nanc, inv)
            g1 = (bi * ui + bj * uj + bk * uk) * inv
            g2 = (ci * ui + cj * uj + ck * uk) * inv
            vi = -(bi * g1 + ci * g2)
            vj = -(bj * g1 + cj * g2)
            vk = -(bk * g1 + ck * g2)
            r = lane_iota + s * LANES
            valid = (tile_base + ch * CHUNK + r) < n_cells
            vi = jnp.where(valid, vi, zeroc)
            vj = jnp.where(valid, vj, zeroc)
            vk = jnp.where(valid, vk, zeroc)
            vbuf[p, 0, sl] = vi
            vbuf[p, 1, sl] = vj
            vbuf[p, 2, sl] = vk

        pltpu.async_copy(vbuf.at[p, 0], acc_sh.at[idx_v.at[0, ch]], ssem, add=True)
        pltpu.async_copy(vbuf.at[p, 1], acc_sh.at[idx_v.at[1, ch]], ssem, add=True)
        pltpu.async_copy(vbuf.at[p, 2], acc_sh.at[idx_v.at[2, ch]], ssem, add=True)
        return carry

    lax.fori_loop(0, n_chunks, chunk_body, 0, unroll=False)
    drain_gathers(n_chunks % 2)
    drain_scatters((n_chunks - 2) % 2)
    drain_scatters((n_chunks - 1) % 2)

    plsc.subcore_barrier()
    pltpu.sync_copy(acc_sh.at[pl.ds(sid * slice_n, slice_n)], out_buf)
    pltpu.sync_copy(out_buf, out_hbm.at[cid, pl.ds(sid * slice_n, slice_n)])


def _build_sc_call(n_cells, cpt, n_chunks):
    mesh = plsc.VectorSubcoreMesh(core_axis_name="c", subcore_axis_name="s")
    slice_n = N_NODES_PAD // 16
    body = functools.partial(_fem_body, n_cells=n_cells, cpt=cpt,
                             n_chunks=n_chunks)
    return pl.kernel(
        body,
        out_type=jax.ShapeDtypeStruct((2, N_NODES_PAD), jnp.float32),
        mesh=mesh,
        scratch_types=dict(
            idx_v=pltpu.VMEM((3, n_chunks + 1, CHUNK), jnp.int32),
            gbuf=pltpu.VMEM((2, 3, 3, CHUNK), jnp.float32),
            vbuf=pltpu.VMEM((2, 3, CHUNK), jnp.float32),
            out_buf=pltpu.VMEM((slice_n,), jnp.float32),
            acc_sh=pltpu.VMEM_SHARED((N_NODES_PAD,), jnp.float32),
            xs_sh=pltpu.VMEM_SHARED((N_NODES_PAD,), jnp.float32),
            ys_sh=pltpu.VMEM_SHARED((N_NODES_PAD,), jnp.float32),
            us_sh=pltpu.VMEM_SHARED((N_NODES_PAD,), jnp.float32),
            sem=pltpu.SemaphoreType.DMA,
            ssem=pltpu.SemaphoreType.DMA,
        ),
    )


def kernel(mesh_points, u, cell_node_map):
    n = mesh_points.shape[0]
    t = cell_node_map.shape[0]
    n_tiles = 32
    cpt = -(-t // (n_tiles * CHUNK)) * CHUNK
    n_chunks = cpt // CHUNK
    slice_n = N_NODES_PAD // 16

    mp = mesh_points.astype(jnp.float32)
    npad = N_NODES_PAD - n
    xs = jnp.pad(mp[:, 0], (0, npad))
    ys = jnp.pad(mp[:, 1], (0, npad))
    us = jnp.pad(u.astype(jnp.float32), (0, npad))

    idx = cell_node_map.astype(jnp.int32).T
    pad = n_tiles * cpt - t
    idx = jnp.pad(idx, ((0, 0), (0, pad)))
    idx = idx.reshape(3, n_tiles, n_chunks, CHUNK)
    idx = jnp.concatenate(
        [idx, jnp.zeros((3, n_tiles, 1, CHUNK), jnp.int32)], axis=2)
    zeros = jnp.zeros((slice_n,), jnp.float32)
    dummy = jnp.zeros((3, 3, CHUNK), jnp.float32)

    call = _build_sc_call(t, cpt, n_chunks)
    partials = call(xs, ys, us, idx, zeros, dummy)
    out = partials[0] + partials[1]
    return out[:n]
